# Initial kernel scaffold; baseline (speedup 1.0000x reference)
import jax
import jax.numpy as jnp
from jax import lax
from jax.experimental import pallas as pl
from jax.experimental.pallas import tpu as pltpu
from jax.experimental.pallas import tpu_sc as plsc

P = 28
PP = 32
N = 10000
E = 160000
F = 128
G = 64
EPS = 1e-5

NC = 2
NS = 16
NW = NC * NS
EW = E // NW
CH = 128
NFULL = EW // CH
TAIL = EW - NFULL * CH
NT = N // NS


def _leaky(v):
    return jnp.where(v >= 0, v, 0.01 * v)


def _pad2(w, rows, cols):
    r, c = w.shape
    return jnp.pad(w, ((0, rows - r), (0, cols - c)))


def _lin0_body(x_ref, w_ref, g_ref, b_ref, o_ref):
    y = jnp.dot(x_ref[...], w_ref[...], preferred_element_type=jnp.float32)
    m = jnp.mean(y, axis=0, keepdims=True)
    v = jnp.mean(y * y, axis=0, keepdims=True) - m * m
    o_ref[...] = _leaky((y - m) * jax.lax.rsqrt(v + EPS) * g_ref[...] + b_ref[...])


def _mom1_body(ea_ref, w1_ref, s_ref, q_ref):
    @pl.when(pl.program_id(0) == 0)
    def _():
        s_ref[...] = jnp.zeros_like(s_ref)
        q_ref[...] = jnp.zeros_like(q_ref)
    y = jnp.dot(ea_ref[...], w1_ref[...], preferred_element_type=jnp.float32)
    s_ref[...] += jnp.sum(y, axis=0, keepdims=True)
    q_ref[...] += jnp.sum(y * y, axis=0, keepdims=True)


def _edget_body(ea_ref, w1_ref, s_ref, q_ref, g_ref, b_ref,
                t_ref, st_ref, tt_ref):
    @pl.when(pl.program_id(0) == 0)
    def _():
        st_ref[...] = jnp.zeros_like(st_ref)
        tt_ref[...] = jnp.zeros_like(tt_ref)
    m1 = s_ref[...] / E
    v1 = q_ref[...] / E - m1 * m1
    y = jnp.dot(ea_ref[...], w1_ref[...], preferred_element_type=jnp.float32)
    t = _leaky((y - m1) * jax.lax.rsqrt(v1 + EPS) * g_ref[...] + b_ref[...])
    t_ref[...] = t
    st_ref[...] += jnp.sum(t, axis=0, keepdims=True)
    tt_ref[...] += lax.dot_general(t, t, (((0,), (0,)), ((), ())),
                                   preferred_element_type=jnp.float32)


def _coeff_body(st_ref, tt_ref, w2_ref, g2_ref, b2_ref, w2a_ref, bet_ref):
    mt = st_ref[...] / E
    m2 = jnp.dot(mt, w2_ref[...], preferred_element_type=jnp.float32)
    c = tt_ref[...] / E - mt.T * mt
    cw = jnp.dot(c, w2_ref[...], preferred_element_type=jnp.float32)
    v2 = jnp.sum(cw * w2_ref[...], axis=0, keepdims=True)
    alpha = g2_ref[...] * jax.lax.rsqrt(v2 + EPS)
    w2a_ref[...] = w2_ref[...] * alpha
    bet_ref[...] = b2_ref[...] - m2 * alpha


def _msg_body(t_ref, hs_ref, a_ref, bm_ref, o_ref):
    t = t_ref[...]
    hs = hs_ref[...]
    z = (t[:, :, None] * hs[:, None, :]).reshape(t.shape[0], PP * PP)
    msg = jnp.dot(z, a_ref[...], preferred_element_type=jnp.float32)
    msg += jnp.dot(hs, bm_ref[...], preferred_element_type=jnp.float32)
    lane = lax.broadcasted_iota(jnp.int32, msg.shape, 1)
    o_ref[...] = msg + (lane == (PP - 1)).astype(jnp.float32)


def _gru_body(p_ref, h_ref, root_ref, cb_ref, wih_ref, bih_ref,
              whh_ref, bhh_ref, o_ref):
    p = p_ref[0] + p_ref[1]
    deg = jnp.clip(p[:, PP - 1:PP], 1.0, None)
    aggr = p / deg
    h = h_ref[...]
    m = _leaky(aggr + jnp.dot(h, root_ref[...], preferred_element_type=jnp.float32)
               + cb_ref[...])
    gi = jnp.dot(m, wih_ref[...], preferred_element_type=jnp.float32) + bih_ref[...]
    gh = jnp.dot(h, whh_ref[...], preferred_element_type=jnp.float32) + bhh_ref[...]
    r = jax.nn.sigmoid(gi[:, 0:PP] + gh[:, 0:PP])
    z = jax.nn.sigmoid(gi[:, PP:2 * PP] + gh[:, PP:2 * PP])
    n = jnp.tanh(gi[:, 2 * PP:3 * PP] + r * gh[:, 2 * PP:3 * PP])
    o_ref[...] = (1.0 - z) * n + z * h


def _s2s_body(out_ref, batch_ref, wih_ref, bih_ref, whh_ref, bhh_ref,
              l1w_ref, l1b_ref, l2w_ref, l2b_ref, lfw_ref, lfb_ref, y_ref):
    out = out_ref[...]
    gid = lax.broadcasted_iota(jnp.int32, (N, G), 1)
    msk = (batch_ref[...] == gid).astype(jnp.float32)
    hl = jnp.zeros((G, PP), jnp.float32)
    cl = jnp.zeros((G, PP), jnp.float32)
    qs = jnp.zeros((G, 2 * PP), jnp.float32)
    for _ in range(3):
        gates = (jnp.dot(qs, wih_ref[...], preferred_element_type=jnp.float32)
                 + bih_ref[...]
                 + jnp.dot(hl, whh_ref[...], preferred_element_type=jnp.float32)
                 + bhh_ref[...])
        i_ = jax.nn.sigmoid(gates[:, 0:PP])
        f_ = jax.nn.sigmoid(gates[:, PP:2 * PP])
        g_ = jnp.tanh(gates[:, 2 * PP:3 * PP])
        o_ = jax.nn.sigmoid(gates[:, 3 * PP:4 * PP])
        cl = f_ * cl + i_ * g_
        hl = o_ * jnp.tanh(cl)
        s = lax.dot_general(out, hl, (((1,), (1,)), ((), ())),
                            preferred_element_type=jnp.float32)
        smask = jnp.where(msk > 0, s, -1e30)
        emax = jnp.max(smask, axis=0, keepdims=True)
        eh = jnp.exp(jnp.where(msk > 0, s - emax, -60.0))
        esum = jnp.sum(eh, axis=0, keepdims=True)
        amat = eh / (esum + 1e-16)
        r_ = lax.dot_general(amat, out, (((0,), (0,)), ((), ())),
                             preferred_element_type=jnp.float32)
        qs = jnp.concatenate([hl, r_], axis=1)
    y1 = _leaky(jnp.dot(qs, l1w_ref[...], preferred_element_type=jnp.float32)
                + l1b_ref[...])
    y2 = _leaky(jnp.dot(y1, l2w_ref[...], preferred_element_type=jnp.float32)
                + l2b_ref[...])
    y_ref[...] = jnp.sum(y2 * lfw_ref[...], axis=1, keepdims=True) + lfb_ref[...]


def _gather_body(h_hbm, src_hbm, out_hbm, idx_v, rows_v, idx_t, rows_t, sem):
    wid = lax.axis_index("s") * NC + lax.axis_index("c")
    base = wid * EW

    def chunk(i, _):
        off = base + i * CH
        pltpu.sync_copy(src_hbm.at[pl.ds(off, CH)], idx_v)
        pltpu.async_copy(h_hbm.at[idx_v], rows_v, sem).wait()
        pltpu.sync_copy(rows_v, out_hbm.at[pl.ds(off, CH)])
        return 0

    lax.fori_loop(0, NFULL, chunk, 0)
    off = base + NFULL * CH
    pltpu.sync_copy(src_hbm.at[pl.ds(off, TAIL)], idx_t)
    pltpu.async_copy(h_hbm.at[idx_t], rows_t, sem).wait()
    pltpu.sync_copy(rows_t, out_hbm.at[pl.ds(off, TAIL)])


def _scatter_body(msg_hbm, dst_hbm, zeros_hbm, out_hbm,
                  idx_v, rows_v, idx_t, rows_t, zrow_v, acc_sh, sem):
    cid = lax.axis_index("c")
    sid = lax.axis_index("s")
    wid = sid * NC + cid
    pltpu.sync_copy(zeros_hbm.at[pl.ds(sid * NT, NT)], zrow_v)
    pltpu.sync_copy(zrow_v, acc_sh.at[pl.ds(sid * NT, NT)])
    plsc.subcore_barrier()

    base = wid * EW

    def chunk(i, _):
        off = base + i * CH
        pltpu.sync_copy(dst_hbm.at[pl.ds(off, CH)], idx_v)
        pltpu.sync_copy(msg_hbm.at[pl.ds(off, CH)], rows_v)
        pltpu.sync_copy(rows_v, acc_sh.at[idx_v], add=True)
        return 0

    lax.fori_loop(0, NFULL, chunk, 0)
    off = base + NFULL * CH
    pltpu.sync_copy(dst_hbm.at[pl.ds(off, TAIL)], idx_t)
    pltpu.sync_copy(msg_hbm.at[pl.ds(off, TAIL)], rows_t)
    pltpu.sync_copy(rows_t, acc_sh.at[idx_t], add=True)

    plsc.subcore_barrier()
    pltpu.sync_copy(acc_sh.at[pl.ds(sid * NT, NT)],
                    out_hbm.at[cid, pl.ds(sid * NT, NT)])


def _sc_gather(h, src):
    mesh = plsc.VectorSubcoreMesh(core_axis_name="c", subcore_axis_name="s",
                                  num_cores=NC, num_subcores=NS)
    return pl.kernel(
        _gather_body,
        out_type=jax.ShapeDtypeStruct((E, PP), jnp.float32),
        mesh=mesh,
        scratch_types=[
            pltpu.VMEM((CH,), jnp.int32),
            pltpu.VMEM((CH, PP), jnp.float32),
            pltpu.VMEM((TAIL,), jnp.int32),
            pltpu.VMEM((TAIL, PP), jnp.float32),
            pltpu.SemaphoreType.DMA,
        ],
    )(h, src)


def _sc_scatter(msg, dst, zeros):
    mesh = plsc.VectorSubcoreMesh(core_axis_name="c", subcore_axis_name="s",
                                  num_cores=NC, num_subcores=NS)
    return pl.kernel(
        _scatter_body,
        out_type=jax.ShapeDtypeStruct((NC, N, PP), jnp.float32),
        mesh=mesh,
        scratch_types=[
            pltpu.VMEM((CH,), jnp.int32),
            pltpu.VMEM((CH, PP), jnp.float32),
            pltpu.VMEM((TAIL,), jnp.int32),
            pltpu.VMEM((TAIL, PP), jnp.float32),
            pltpu.VMEM((NT, PP), jnp.float32),
            pltpu.VMEM_SHARED((N, PP), jnp.float32),
            pltpu.SemaphoreType.DMA,
        ],
    )(msg, dst, zeros)


_MB = 1000
_NB = E // _MB


def kernel(x, edge_index, edge_attr, batch, lin0_W, bn0_g, bn0_b, enn_W1,
           enn_bn1_g, enn_bn1_b, enn_W2, enn_bn2_g, enn_bn2_b, conv_root,
           conv_bias, gru_Wih, gru_bih, gru_Whh, gru_bhh, s2s_Wih, s2s_bih,
           s2s_Whh, s2s_bhh, lin1_W, lin1_b, lin2_W, lin2_b, linf_W, linf_b):
    f32 = jnp.float32
    src = edge_index[0].astype(jnp.int32)
    dst = edge_index[1].astype(jnp.int32)

    lin0p = _pad2(lin0_W, F, PP)
    g0p = _pad2(bn0_g[None, :], 1, PP)
    b0p = _pad2(bn0_b[None, :], 1, PP)
    w1p = _pad2(enn_W1, 4, PP)
    g1p = _pad2(enn_bn1_g[None, :], 1, PP)
    b1p = _pad2(enn_bn1_b[None, :], 1, PP)
    w2p = _pad2(enn_W2, PP, P * P)
    g2p = enn_bn2_g[None, :]
    b2p = enn_bn2_b[None, :]
    rootp = _pad2(conv_root, PP, PP)
    cbp = _pad2(conv_bias[None, :], 1, PP)

    def gate_pad(w, ngate, rpad):
        return jnp.pad(w.reshape(w.shape[0], ngate, P),
                       ((0, rpad - w.shape[0]), (0, 0), (0, PP - P))
                       ).reshape(rpad, ngate * PP)

    wihp = gate_pad(gru_Wih, 3, PP)
    bihp = gate_pad(gru_bih[None, :], 3, 1)
    whhp = gate_pad(gru_Whh, 3, PP)
    bhhp = gate_pad(gru_bhh[None, :], 3, 1)
    s2s_wihp = jnp.pad(s2s_Wih.reshape(2, P, 4, P),
                       ((0, 0), (0, PP - P), (0, 0), (0, PP - P))
                       ).reshape(2 * PP, 4 * PP)
    s2s_bihp = gate_pad(s2s_bih[None, :], 4, 1)
    s2s_whhp = gate_pad(s2s_Whh, 4, PP)
    s2s_bhhp = gate_pad(s2s_bhh[None, :], 4, 1)
    l1wp = jnp.pad(lin1_W.reshape(2, P, P),
                   ((0, 0), (0, PP - P), (0, PP - P))).reshape(2 * PP, PP)
    l1bp = _pad2(lin1_b[None, :], 1, PP)
    l2wp = _pad2(lin2_W, PP, 16)
    l2bp = _pad2(lin2_b[None, :], 1, 16)
    lfwp = _pad2(linf_W.T, 1, 16)
    lfbp = linf_b[None, :]
    batch2d = batch.astype(jnp.int32)[:, None]
    zeros_n = jnp.zeros((N, PP), f32)

    h0 = pl.pallas_call(
        _lin0_body,
        out_shape=jax.ShapeDtypeStruct((N, PP), f32),
    )(x, lin0p, g0p, b0p)

    eb = 8000
    neb = E // eb
    s1, q1 = pl.pallas_call(
        _mom1_body,
        grid=(neb,),
        in_specs=[pl.BlockSpec((eb, 4), lambda i: (i, 0)),
                  pl.BlockSpec((4, PP), lambda i: (0, 0))],
        out_specs=[pl.BlockSpec((1, PP), lambda i: (0, 0)),
                   pl.BlockSpec((1, PP), lambda i: (0, 0))],
        out_shape=[jax.ShapeDtypeStruct((1, PP), f32),
                   jax.ShapeDtypeStruct((1, PP), f32)],
    )(edge_attr, w1p)

    t, st, ttt = pl.pallas_call(
        _edget_body,
        grid=(neb,),
        in_specs=[pl.BlockSpec((eb, 4), lambda i: (i, 0)),
                  pl.BlockSpec((4, PP), lambda i: (0, 0)),
                  pl.BlockSpec((1, PP), lambda i: (0, 0)),
                  pl.BlockSpec((1, PP), lambda i: (0, 0)),
                  pl.BlockSpec((1, PP), lambda i: (0, 0)),
                  pl.BlockSpec((1, PP), lambda i: (0, 0))],
        out_specs=[pl.BlockSpec((eb, PP), lambda i: (i, 0)),
                   pl.BlockSpec((1, PP), lambda i: (0, 0)),
                   pl.BlockSpec((PP, PP), lambda i: (0, 0))],
        out_shape=[jax.ShapeDtypeStruct((E, PP), f32),
                   jax.ShapeDtypeStruct((1, PP), f32),
                   jax.ShapeDtypeStruct((PP, PP), f32)],
    )(edge_attr, w1p, s1, q1, g1p, b1p)

    w2a, bet2 = pl.pallas_call(
        _coeff_body,
        out_shape=[jax.ShapeDtypeStruct((PP, P * P), f32),
                   jax.ShapeDtypeStruct((1, P * P), f32)],
    )(st, ttt, w2p, g2p, b2p)

    a_big = jnp.pad(w2a[:P].reshape(P, P, P),
                    ((0, PP - P), (0, PP - P), (0, PP - P))
                    ).reshape(PP * PP, PP)
    bm_pad = jnp.pad(bet2[0].reshape(P, P), ((0, PP - P), (0, PP - P)))

    msg_call = pl.pallas_call(
        _msg_body,
        grid=(_NB,),
        in_specs=[pl.BlockSpec((_MB, PP), lambda i: (i, 0)),
                  pl.BlockSpec((_MB, PP), lambda i: (i, 0)),
                  pl.BlockSpec((PP * PP, PP), lambda i: (0, 0)),
                  pl.BlockSpec((PP, PP), lambda i: (0, 0))],
        out_specs=pl.BlockSpec((_MB, PP), lambda i: (i, 0)),
        out_shape=jax.ShapeDtypeStruct((E, PP), f32),
    )

    gru_call = pl.pallas_call(
        _gru_body,
        out_shape=jax.ShapeDtypeStruct((N, PP), f32),
    )

    h = h0
    for _ in range(3):
        hs = _sc_gather(h, src)
        msg = msg_call(t, hs, a_big, bm_pad)
        parts = _sc_scatter(msg, dst, zeros_n)
        h = gru_call(parts, h, rootp, cbp, wihp, bihp, whhp, bhhp)

    y = pl.pallas_call(
        _s2s_body,
        out_shape=jax.ShapeDtypeStruct((G, 1), f32),
    )(h, batch2d, s2s_wihp, s2s_bihp, s2s_whhp, s2s_bhhp,
      l1wp, l1bp, l2wp, l2bp, lfwp, lfbp)
    return y[:, 0]


# speedup vs baseline: 1.6376x; 1.6376x over previous
import jax
import jax.numpy as jnp
from jax import lax
from jax.experimental import pallas as pl
from jax.experimental.pallas import tpu as pltpu
from jax.experimental.pallas import tpu_sc as plsc

P = 28
PP = 32
N = 10000
E = 160000
F = 128
G = 64
EPS = 1e-5

NC = 2
NS = 16
NW = NC * NS
EW = E // NW
CH = 128
NFULL = EW // CH
TAIL = EW - NFULL * CH
NT = N // NS


def _leaky(v):
    return jnp.where(v >= 0, v, 0.01 * v)


def _bdot(a, b, dims=None):
    a16 = a.astype(jnp.bfloat16)
    b16 = b.astype(jnp.bfloat16)
    if dims is None:
        dims = (((a.ndim - 1,), (0,)), ((), ()))
    return lax.dot_general(a16, b16, dims, preferred_element_type=jnp.float32)


def _hdot(a, b, dims=None):
    if dims is None:
        dims = (((a.ndim - 1,), (0,)), ((), ()))
    return lax.dot_general(a, b, dims, preferred_element_type=jnp.float32,
                           precision=lax.Precision.HIGHEST)


def _pad2(w, rows, cols):
    r, c = w.shape
    return jnp.pad(w, ((0, rows - r), (0, cols - c)))


def _lin0_body(x_ref, w_ref, g_ref, b_ref, o_ref):
    y = _bdot(x_ref[...], w_ref[...])
    m = jnp.mean(y, axis=0, keepdims=True)
    v = jnp.mean(y * y, axis=0, keepdims=True) - m * m
    o_ref[...] = _leaky((y - m) * jax.lax.rsqrt(v + EPS) * g_ref[...] + b_ref[...])


def _mom1_body(ea_ref, w1_ref, s_ref, q_ref):
    @pl.when(pl.program_id(0) == 0)
    def _():
        s_ref[...] = jnp.zeros_like(s_ref)
        q_ref[...] = jnp.zeros_like(q_ref)
    y = _bdot(ea_ref[...], w1_ref[...])
    s_ref[...] += jnp.sum(y, axis=0, keepdims=True)
    q_ref[...] += jnp.sum(y * y, axis=0, keepdims=True)


def _edget_body(ea_ref, w1_ref, s_ref, q_ref, g_ref, b_ref,
                t_ref, st_ref, tt_ref):
    @pl.when(pl.program_id(0) == 0)
    def _():
        st_ref[...] = jnp.zeros_like(st_ref)
        tt_ref[...] = jnp.zeros_like(tt_ref)
    m1 = s_ref[...] / E
    v1 = q_ref[...] / E - m1 * m1
    y = _bdot(ea_ref[...], w1_ref[...])
    t = _leaky((y - m1) * jax.lax.rsqrt(v1 + EPS) * g_ref[...] + b_ref[...])
    t_ref[...] = t
    bt = t.astype(jnp.bfloat16).astype(jnp.float32)
    st_ref[...] += jnp.sum(bt, axis=0, keepdims=True)
    tt_ref[...] += _hdot(bt, bt, (((0,), (0,)), ((), ())))


def _coeff_body(st_ref, tt_ref, w2_ref, g2_ref, b2_ref, alp_ref, bet_ref):
    w2 = w2_ref[...].astype(jnp.float32)
    mt = st_ref[...] / E
    m2 = _hdot(mt, w2)
    c = tt_ref[...] / E - mt.T * mt
    cw = _hdot(c, w2)
    v2 = jnp.sum(cw * w2, axis=0, keepdims=True)
    alpha = g2_ref[...] * jax.lax.rsqrt(v2 + EPS)
    alp_ref[...] = alpha
    bet_ref[...] = b2_ref[...] - m2 * alpha


def _msg_body(t_ref, hs_ref, w2k_ref, a_ref, b_ref, o_ref):
    we = lax.dot_general(t_ref[...].astype(jnp.bfloat16), w2k_ref[...],
                         (((1,), (0,)), ((), ())),
                         preferred_element_type=jnp.float32)
    we = we * a_ref[...] + b_ref[...]
    we = we.astype(jnp.bfloat16).astype(jnp.float32)
    hs = hs_ref[...].astype(jnp.bfloat16).astype(jnp.float32)
    msg = hs[:, 0:1] * we[:, 0:PP]
    for p in range(1, P):
        msg += hs[:, p:p + 1] * we[:, p * PP:(p + 1) * PP]
    lane = lax.broadcasted_iota(jnp.int32, msg.shape, 1)
    o_ref[...] = msg + (lane == (PP - 1)).astype(jnp.float32)


def _gru_body(p_ref, h_ref, root_ref, cb_ref, wih_ref, bih_ref,
              whh_ref, bhh_ref, o_ref):
    p = p_ref[0] + p_ref[1]
    deg = jnp.clip(p[:, PP - 1:PP], 1.0, None)
    aggr = p / deg
    h = h_ref[...]
    m = _leaky(aggr + _bdot(h, root_ref[...]) + cb_ref[...])
    gi = _bdot(m, wih_ref[...]) + bih_ref[...]
    gh = _bdot(h, whh_ref[...]) + bhh_ref[...]
    r = jax.nn.sigmoid(gi[:, 0:PP] + gh[:, 0:PP])
    z = jax.nn.sigmoid(gi[:, PP:2 * PP] + gh[:, PP:2 * PP])
    n = jnp.tanh(gi[:, 2 * PP:3 * PP] + r * gh[:, 2 * PP:3 * PP])
    o_ref[...] = (1.0 - z) * n + z * h


def _s2s_body(out_ref, batch_ref, wih_ref, bih_ref, whh_ref, bhh_ref,
              l1w_ref, l1b_ref, l2w_ref, l2b_ref, lfw_ref, lfb_ref, y_ref):
    out = out_ref[...]
    gid = lax.broadcasted_iota(jnp.int32, (N, G), 1)
    msk = (batch_ref[...] == gid).astype(jnp.float32)
    hl = jnp.zeros((G, PP), jnp.float32)
    cl = jnp.zeros((G, PP), jnp.float32)
    qs = jnp.zeros((G, 2 * PP), jnp.float32)
    for _ in range(3):
        gates = (_bdot(qs, wih_ref[...]) + bih_ref[...]
                 + _bdot(hl, whh_ref[...]) + bhh_ref[...])
        i_ = jax.nn.sigmoid(gates[:, 0:PP])
        f_ = jax.nn.sigmoid(gates[:, PP:2 * PP])
        g_ = jnp.tanh(gates[:, 2 * PP:3 * PP])
        o_ = jax.nn.sigmoid(gates[:, 3 * PP:4 * PP])
        cl = f_ * cl + i_ * g_
        hl = o_ * jnp.tanh(cl)
        s = _hdot(out, hl, (((1,), (1,)), ((), ())))
        smask = jnp.where(msk > 0, s, -1e30)
        emax = jnp.max(smask, axis=0, keepdims=True)
        eh = jnp.exp(jnp.where(msk > 0, s - emax, -60.0))
        esum = jnp.sum(eh, axis=0, keepdims=True)
        amat = eh / (esum + 1e-16)
        r_ = _hdot(amat, out, (((0,), (0,)), ((), ())))
        qs = jnp.concatenate([hl, r_], axis=1)
    y1 = _leaky(_bdot(qs, l1w_ref[...]) + l1b_ref[...])
    y2 = _leaky(_bdot(y1, l2w_ref[...]) + l2b_ref[...])
    b2 = y2.astype(jnp.bfloat16).astype(jnp.float32)
    bw = lfw_ref[...].astype(jnp.bfloat16).astype(jnp.float32)
    y_ref[...] = jnp.sum(b2 * bw, axis=1, keepdims=True) + lfb_ref[...]


def _gather_body(h_hbm, src_hbm, out_hbm, idx_v, rows_v, idx_t, rows_t, sem):
    wid = lax.axis_index("s") * NC + lax.axis_index("c")
    base = wid * EW

    def chunk(i, _):
        off = base + i * CH
        pltpu.sync_copy(src_hbm.at[pl.ds(off, CH)], idx_v)
        pltpu.async_copy(h_hbm.at[idx_v], rows_v, sem).wait()
        pltpu.sync_copy(rows_v, out_hbm.at[pl.ds(off, CH)])
        return 0

    lax.fori_loop(0, NFULL, chunk, 0)
    off = base + NFULL * CH
    pltpu.sync_copy(src_hbm.at[pl.ds(off, TAIL)], idx_t)
    pltpu.async_copy(h_hbm.at[idx_t], rows_t, sem).wait()
    pltpu.sync_copy(rows_t, out_hbm.at[pl.ds(off, TAIL)])


def _scatter_body(msg_hbm, dst_hbm, zeros_hbm, out_hbm,
                  idx_v, rows_v, idx_t, rows_t, zrow_v, acc_sh, sem):
    cid = lax.axis_index("c")
    sid = lax.axis_index("s")
    wid = sid * NC + cid
    pltpu.sync_copy(zeros_hbm.at[pl.ds(sid * NT, NT)], zrow_v)
    pltpu.sync_copy(zrow_v, acc_sh.at[pl.ds(sid * NT, NT)])
    plsc.subcore_barrier()

    base = wid * EW

    def chunk(i, _):
        off = base + i * CH
        pltpu.sync_copy(dst_hbm.at[pl.ds(off, CH)], idx_v)
        pltpu.sync_copy(msg_hbm.at[pl.ds(off, CH)], rows_v)
        pltpu.sync_copy(rows_v, acc_sh.at[idx_v], add=True)
        return 0

    lax.fori_loop(0, NFULL, chunk, 0)
    off = base + NFULL * CH
    pltpu.sync_copy(dst_hbm.at[pl.ds(off, TAIL)], idx_t)
    pltpu.sync_copy(msg_hbm.at[pl.ds(off, TAIL)], rows_t)
    pltpu.sync_copy(rows_t, acc_sh.at[idx_t], add=True)

    plsc.subcore_barrier()
    pltpu.sync_copy(acc_sh.at[pl.ds(sid * NT, NT)],
                    out_hbm.at[cid, pl.ds(sid * NT, NT)])


_SC_PARAMS = pltpu.CompilerParams(use_tc_tiling_on_sc=False)


def _sc_gather(h, src):
    mesh = plsc.VectorSubcoreMesh(core_axis_name="c", subcore_axis_name="s",
                                  num_cores=NC, num_subcores=NS)
    return pl.kernel(
        _gather_body,
        out_type=jax.ShapeDtypeStruct((E, PP), jnp.float32),
        mesh=mesh,
        compiler_params=_SC_PARAMS,
        scratch_types=[
            pltpu.VMEM((CH,), jnp.int32),
            pltpu.VMEM((CH, PP), jnp.float32),
            pltpu.VMEM((TAIL,), jnp.int32),
            pltpu.VMEM((TAIL, PP), jnp.float32),
            pltpu.SemaphoreType.DMA,
        ],
    )(h, src)


def _sc_scatter(msg, dst, zeros):
    mesh = plsc.VectorSubcoreMesh(core_axis_name="c", subcore_axis_name="s",
                                  num_cores=NC, num_subcores=NS)
    return pl.kernel(
        _scatter_body,
        out_type=jax.ShapeDtypeStruct((NC, N, PP), jnp.float32),
        mesh=mesh,
        compiler_params=_SC_PARAMS,
        scratch_types=[
            pltpu.VMEM((CH,), jnp.int32),
            pltpu.VMEM((CH, PP), jnp.float32),
            pltpu.VMEM((TAIL,), jnp.int32),
            pltpu.VMEM((TAIL, PP), jnp.float32),
            pltpu.VMEM((NT, PP), jnp.float32),
            pltpu.VMEM_SHARED((N, PP), jnp.float32),
            pltpu.SemaphoreType.DMA,
        ],
    )(msg, dst, zeros)


_MB = 1000
_NB = E // _MB


def kernel(x, edge_index, edge_attr, batch, lin0_W, bn0_g, bn0_b, enn_W1,
           enn_bn1_g, enn_bn1_b, enn_W2, enn_bn2_g, enn_bn2_b, conv_root,
           conv_bias, gru_Wih, gru_bih, gru_Whh, gru_bhh, s2s_Wih, s2s_bih,
           s2s_Whh, s2s_bhh, lin1_W, lin1_b, lin2_W, lin2_b, linf_W, linf_b):
    f32 = jnp.float32
    src = edge_index[0].astype(jnp.int32)
    dst = edge_index[1].astype(jnp.int32)

    lin0p = _pad2(lin0_W, F, PP)
    g0p = _pad2(bn0_g[None, :], 1, PP)
    b0p = _pad2(bn0_b[None, :], 1, PP)
    w1p = _pad2(enn_W1, 4, PP)
    g1p = _pad2(enn_bn1_g[None, :], 1, PP)
    b1p = _pad2(enn_bn1_b[None, :], 1, PP)
    w2b = _pad2(enn_W2, PP, P * P).astype(jnp.bfloat16)
    w2k = jnp.pad(enn_W2.reshape(P, P, P),
                  ((0, PP - P), (0, PP - P), (0, PP - P))
                  ).reshape(PP, PP * PP).astype(jnp.bfloat16)
    g2p = enn_bn2_g[None, :]
    b2p = enn_bn2_b[None, :]
    rootp = _pad2(conv_root, PP, PP)
    cbp = _pad2(conv_bias[None, :], 1, PP)

    def gate_pad(w, ngate, rpad):
        return jnp.pad(w.reshape(w.shape[0], ngate, P),
                       ((0, rpad - w.shape[0]), (0, 0), (0, PP - P))
                       ).reshape(rpad, ngate * PP)

    wihp = gate_pad(gru_Wih, 3, PP)
    bihp = gate_pad(gru_bih[None, :], 3, 1)
    whhp = gate_pad(gru_Whh, 3, PP)
    bhhp = gate_pad(gru_bhh[None, :], 3, 1)
    s2s_wihp = jnp.pad(s2s_Wih.reshape(2, P, 4, P),
                       ((0, 0), (0, PP - P), (0, 0), (0, PP - P))
                       ).reshape(2 * PP, 4 * PP)
    s2s_bihp = gate_pad(s2s_bih[None, :], 4, 1)
    s2s_whhp = gate_pad(s2s_Whh, 4, PP)
    s2s_bhhp = gate_pad(s2s_bhh[None, :], 4, 1)
    l1wp = jnp.pad(lin1_W.reshape(2, P, P),
                   ((0, 0), (0, PP - P), (0, PP - P))).reshape(2 * PP, PP)
    l1bp = _pad2(lin1_b[None, :], 1, PP)
    l2wp = _pad2(lin2_W, PP, 16)
    l2bp = _pad2(lin2_b[None, :], 1, 16)
    lfwp = _pad2(linf_W.T, 1, 16)
    lfbp = linf_b[None, :]
    batch2d = batch.astype(jnp.int32)[:, None]
    zeros_n = jnp.zeros((N, PP), f32)

    h0 = pl.pallas_call(
        _lin0_body,
        out_shape=jax.ShapeDtypeStruct((N, PP), f32),
    )(x, lin0p, g0p, b0p)

    eb = 8000
    neb = E // eb
    s1, q1 = pl.pallas_call(
        _mom1_body,
        grid=(neb,),
        in_specs=[pl.BlockSpec((eb, 4), lambda i: (i, 0)),
                  pl.BlockSpec((4, PP), lambda i: (0, 0))],
        out_specs=[pl.BlockSpec((1, PP), lambda i: (0, 0)),
                   pl.BlockSpec((1, PP), lambda i: (0, 0))],
        out_shape=[jax.ShapeDtypeStruct((1, PP), f32),
                   jax.ShapeDtypeStruct((1, PP), f32)],
    )(edge_attr, w1p)

    t, st, ttt = pl.pallas_call(
        _edget_body,
        grid=(neb,),
        in_specs=[pl.BlockSpec((eb, 4), lambda i: (i, 0)),
                  pl.BlockSpec((4, PP), lambda i: (0, 0)),
                  pl.BlockSpec((1, PP), lambda i: (0, 0)),
                  pl.BlockSpec((1, PP), lambda i: (0, 0)),
                  pl.BlockSpec((1, PP), lambda i: (0, 0)),
                  pl.BlockSpec((1, PP), lambda i: (0, 0))],
        out_specs=[pl.BlockSpec((eb, PP), lambda i: (i, 0)),
                   pl.BlockSpec((1, PP), lambda i: (0, 0)),
                   pl.BlockSpec((PP, PP), lambda i: (0, 0))],
        out_shape=[jax.ShapeDtypeStruct((E, PP), f32),
                   jax.ShapeDtypeStruct((1, PP), f32),
                   jax.ShapeDtypeStruct((PP, PP), f32)],
    )(edge_attr, w1p, s1, q1, g1p, b1p)

    alp2, bet2 = pl.pallas_call(
        _coeff_body,
        out_shape=[jax.ShapeDtypeStruct((1, P * P), f32),
                   jax.ShapeDtypeStruct((1, P * P), f32)],
    )(st, ttt, w2b, g2p, b2p)

    a1024 = jnp.pad(alp2[0].reshape(P, P),
                    ((0, PP - P), (0, PP - P))).reshape(1, PP * PP)
    b1024 = jnp.pad(bet2[0].reshape(P, P),
                    ((0, PP - P), (0, PP - P))).reshape(1, PP * PP)

    msg_call = pl.pallas_call(
        _msg_body,
        grid=(_NB,),
        in_specs=[pl.BlockSpec((_MB, PP), lambda i: (i, 0)),
                  pl.BlockSpec((_MB, PP), lambda i: (i, 0)),
                  pl.BlockSpec((PP, PP * PP), lambda i: (0, 0)),
                  pl.BlockSpec((1, PP * PP), lambda i: (0, 0)),
                  pl.BlockSpec((1, PP * PP), lambda i: (0, 0))],
        out_specs=pl.BlockSpec((_MB, PP), lambda i: (i, 0)),
        out_shape=jax.ShapeDtypeStruct((E, PP), f32),
    )

    gru_call = pl.pallas_call(
        _gru_body,
        out_shape=jax.ShapeDtypeStruct((N, PP), f32),
    )

    h = h0
    for _ in range(3):
        hs = _sc_gather(h, src)
        msg = msg_call(t, hs, w2k, a1024, b1024)
        parts = _sc_scatter(msg, dst, zeros_n)
        h = gru_call(parts, h, rootp, cbp, wihp, bihp, whhp, bhhp)

    y = pl.pallas_call(
        _s2s_body,
        out_shape=jax.ShapeDtypeStruct((G, 1), f32),
    )(h, batch2d, s2s_wihp, s2s_bihp, s2s_whhp, s2s_bhhp,
      l1wp, l1bp, l2wp, l2bp, lfwp, lfbp)
    return y[:, 0]
